# baseline (speedup 1.0000x reference)
import functools

import jax
import jax.numpy as jnp
from jax import lax
from jax.experimental import pallas as pl
from jax.experimental.pallas import tpu as pltpu
from jax.experimental.pallas import tpu_sc as plsc

R_CUT = 5.0
N_RBF = 16
SPHC_NORM = 32.0

_C0 = 0.28209479177387814
_C1 = 0.4886025119029199
_C2A = 1.0925484305920792
_C2B = 0.31539156525252005
_C2C = 0.5462742152960396

_COS_COEF = (0.9999999922898466, -4.934801388370915, 4.058698262269046,
             -1.335174453410593, 0.2350633717632637, -0.02539111138456386,
             0.001605362776714614)

_EDGE_BLOCK = 512


def _tc_body(r_ref, sph_ref, g_ref):
    r = r_ref[...]
    x = r[:, 0:1]
    y = r[:, 1:2]
    z = r[:, 2:3]
    d2 = x * x + y * y + z * z
    inv = lax.rsqrt(d2)
    d = d2 * inv
    ux = x * inv
    uy = y * inv
    uz = z * inv
    phi = jnp.where(d < R_CUT, 0.5 * (jnp.cos((jnp.pi / R_CUT) * d) + 1.0), 0.0)
    mu = (R_CUT / (N_RBF - 1)) * lax.broadcasted_iota(
        jnp.int32, (1, N_RBF), 1).astype(jnp.float32)
    rbf_cut = jnp.exp(-10.0 * (d - mu) ** 2) * phi
    sph = jnp.concatenate([
        jnp.full_like(ux, _C0),
        _C1 * ux, _C1 * uy, _C1 * uz,
        _C2A * ux * uy, _C2A * uy * uz, _C2B * (3.0 * uz * uz - 1.0),
        _C2A * ux * uz, _C2C * (ux * ux - uy * uy),
    ], axis=1)
    sph_ref[...] = sph
    g_ref[...] = sph[:, :, None] * rbf_cut[:, None, :]


def _tc_main(r, n_edges):
    b = _EDGE_BLOCK
    grid = n_edges // b
    return pl.pallas_call(
        _tc_body,
        grid=(grid,),
        in_specs=[pl.BlockSpec((b, 3), lambda i: (i, 0))],
        out_specs=[
            pl.BlockSpec((b, 9), lambda i: (i, 0)),
            pl.BlockSpec((b, 9, 16), lambda i: (i, 0, 0)),
        ],
        out_shape=[
            jax.ShapeDtypeStruct((n_edges, 9), jnp.float32),
            jax.ShapeDtypeStruct((n_edges, 9, 16), jnp.float32),
        ],
    )(r)


def _edge_weights(x, y, z):
    d2 = x * x + y * y + z * z
    i = plsc.bitcast(d2, jnp.int32)
    i = 0x5F3759DF - (i >> 1)
    yv = plsc.bitcast(i, jnp.float32)
    for _ in range(3):
        yv = yv * (1.5 - 0.5 * d2 * yv * yv)
    inv = yv
    ux = x * inv
    uy = y * inv
    uz = z * inv
    s = d2 * (1.0 / (R_CUT * R_CUT))
    c = jnp.float32(_COS_COEF[6])
    for k in (5, 4, 3, 2, 1, 0):
        c = c * s + jnp.float32(_COS_COEF[k])
    phi = jnp.where(s < 1.0, 0.5 * (c + 1.0), jnp.float32(0.0))
    return (
        _C0 * phi,
        (_C1 * phi) * ux, (_C1 * phi) * uy, (_C1 * phi) * uz,
        (_C2A * phi) * ux * uy, (_C2A * phi) * uy * uz,
        (_C2B * phi) * (3.0 * uz * uz - 1.0), (_C2A * phi) * ux * uz,
        (_C2C * phi) * (ux * ux - uy * uy),
    )


def _chi_sc_call(r2d, idx2d, n_nodes, n_edges):
    rows_per_sub = ((n_nodes + 16 * 8 - 1) // (16 * 8)) * 8
    n_pad = rows_per_sub * 16
    mesh = plsc.VectorSubcoreMesh(core_axis_name="c", subcore_axis_name="s")
    n_full = n_edges // 1024
    tail_edges = n_edges - n_full * 1024
    tail_inner = tail_edges // 128
    n_chunks = n_full + (1 if tail_edges else 0)
    iters = (n_chunks + 31) // 32

    @functools.partial(
        pl.kernel,
        mesh=mesh,
        out_type=jax.ShapeDtypeStruct((2, n_pad, 16), jnp.float32),
        scratch_types=[
            pltpu.VMEM((8, 128), jnp.int32),
            pltpu.VMEM((24, 128), jnp.float32),
            pltpu.VMEM((128, 16), jnp.float32),
            pltpu.VMEM((rows_per_sub, 16), jnp.float32),
            pltpu.VMEM_SHARED((n_pad, 16), jnp.float32),
        ],
        compiler_params=pltpu.CompilerParams(needs_layout_passes=False),
    )
    def chi_kernel(r_hbm, idx_hbm, out_hbm, idxv, rv, sv, tmp, acc):
        cid = lax.axis_index("c")
        sid = lax.axis_index("s")
        wid = sid * 2 + cid

        zero16 = jnp.zeros((16,), jnp.float32)

        def zrow(i, _):
            tmp[i, :] = zero16
            return 0

        lax.fori_loop(0, rows_per_sub, zrow, 0)

        def zrow2(i, _):
            sv[i, :] = zero16
            return 0

        lax.fori_loop(0, 128, zrow2, 0)

        r0 = sid * rows_per_sub
        pltpu.sync_copy(tmp, acc.at[pl.ds(r0, rows_per_sub)])
        plsc.subcore_barrier()

        lanes = lax.iota(jnp.int32, 16)

        def inner(k):
            for gidx in range(8):
                el = gidx * 16 + lanes
                p0 = (384 * k) + el * 3
                x = plsc.load_gather(rv, [p0 >> 7, p0 & 127])
                y = plsc.load_gather(rv, [(p0 + 1) >> 7, (p0 + 1) & 127])
                z = plsc.load_gather(rv, [(p0 + 2) >> 7, (p0 + 2) & 127])
                w = _edge_weights(x, y, z)
                for j in range(9):
                    plsc.store_scatter(
                        sv, [el, jnp.full((16,), j, jnp.int32)], w[j])
            pltpu.sync_copy(sv, acc.at[idxv.at[k]], add=True)

        def chunk_body(i, _):
            c = i * 32 + wid

            @pl.when(c < n_full)
            def _():
                pltpu.sync_copy(idx_hbm.at[pl.ds(8 * c, 8)], idxv)
                pltpu.sync_copy(r_hbm.at[pl.ds(24 * c, 24)], rv)
                for k in range(8):
                    inner(k)

            if tail_edges:
                @pl.when(c == n_full)
                def _():
                    pltpu.sync_copy(
                        idx_hbm.at[pl.ds(8 * n_full, tail_edges // 128)],
                        idxv.at[pl.ds(0, tail_edges // 128)])
                    pltpu.sync_copy(
                        r_hbm.at[pl.ds(24 * n_full, 3 * tail_edges // 128)],
                        rv.at[pl.ds(0, 3 * tail_edges // 128)])
                    for k in range(tail_inner):
                        inner(k)

            return 0

        lax.fori_loop(0, iters, chunk_body, 0)

        plsc.subcore_barrier()
        pltpu.sync_copy(acc.at[pl.ds(r0, rows_per_sub)], tmp)
        pltpu.sync_copy(tmp, out_hbm.at[cid, pl.ds(r0, rows_per_sub)])

    return chi_kernel(r2d, idx2d)


def _combine(partials, n_nodes):
    def body(p_ref, chi_ref):
        chi_ref[...] = (
            p_ref[0, :n_nodes, :9] + p_ref[1, :n_nodes, :9]
        ) * (1.0 / SPHC_NORM)

    return pl.pallas_call(
        body,
        out_shape=jax.ShapeDtypeStruct((n_nodes, 9), jnp.float32),
    )(partials)


def kernel(pairwise_distances, idx_i, z):
    n_edges = pairwise_distances.shape[0]
    n_nodes = z.shape[0]
    n_chunks = n_edges // 128

    sph, g = _tc_main(pairwise_distances, n_edges)

    _BISECT_TC_ONLY = True
    if _BISECT_TC_ONLY:
        d_ij = jnp.linalg.norm(pairwise_distances, axis=-1)
        phi = jnp.where(d_ij < R_CUT,
                        0.5 * (jnp.cos(jnp.pi * d_ij / R_CUT) + 1.0), 0.0)
        chi = jax.ops.segment_sum(
            sph * phi[:, None], idx_i, num_segments=n_nodes) / SPHC_NORM
    else:
        r2d = pairwise_distances.reshape(n_edges * 3 // 128, 128)
        idx2d = idx_i.astype(jnp.int32).reshape(n_chunks, 128)
        partials = _chi_sc_call(r2d, idx2d, n_nodes, n_edges)
        chi = _combine(partials, n_nodes)
    return sph, chi, g


# speedup vs baseline: 3.4509x; 3.4509x over previous
import functools

import jax
import jax.numpy as jnp
from jax import lax
from jax.experimental import pallas as pl
from jax.experimental.pallas import tpu as pltpu
from jax.experimental.pallas import tpu_sc as plsc

R_CUT = 5.0
N_RBF = 16
SPHC_NORM = 32.0

_C0 = 0.28209479177387814
_C1 = 0.4886025119029199
_C2A = 1.0925484305920792
_C2B = 0.31539156525252005
_C2C = 0.5462742152960396

_COS_COEF = (0.9999999922898466, -4.934801388370915, 4.058698262269046,
             -1.335174453410593, 0.2350633717632637, -0.02539111138456386,
             0.001605362776714614)

_EDGE_BLOCK = 2560


def _tc_body(rt_ref, sph_ref, g_ref):
    x = rt_ref[0:1, :]
    y = rt_ref[1:2, :]
    zc = rt_ref[2:3, :]
    d2 = x * x + y * y + zc * zc
    inv = lax.rsqrt(d2)
    d = d2 * inv
    ux = x * inv
    uy = y * inv
    uz = zc * inv
    phi = jnp.where(d < R_CUT, 0.5 * (jnp.cos((jnp.pi / R_CUT) * d) + 1.0), 0.0)
    mu = (R_CUT / (N_RBF - 1)) * lax.broadcasted_iota(
        jnp.int32, (N_RBF, 1), 0).astype(jnp.float32)
    rbf_cut = jnp.exp(-10.0 * (d - mu) ** 2) * phi
    sph = jnp.concatenate([
        jnp.full_like(ux, _C0),
        _C1 * ux, _C1 * uy, _C1 * uz,
        _C2A * ux * uy, _C2A * uy * uz, _C2B * (3.0 * uz * uz - 1.0),
        _C2A * ux * uz, _C2C * (ux * ux - uy * uy),
    ], axis=0)
    sph_ref[...] = sph
    g_ref[...] = sph[:, None, :] * rbf_cut[None, :, :]


def _tc_main(rt, n_edges):
    b = _EDGE_BLOCK
    grid = n_edges // b
    return pl.pallas_call(
        _tc_body,
        grid=(grid,),
        in_specs=[pl.BlockSpec((3, b), lambda i: (0, i))],
        out_specs=[
            pl.BlockSpec((9, b), lambda i: (0, i)),
            pl.BlockSpec((9, 16, b), lambda i: (0, 0, i)),
        ],
        out_shape=[
            jax.ShapeDtypeStruct((9, n_edges), jnp.float32),
            jax.ShapeDtypeStruct((9, 16, n_edges), jnp.float32),
        ],
    )(rt)


def _edge_weights(x, y, z):
    d2 = x * x + y * y + z * z
    i = plsc.bitcast(d2, jnp.int32)
    i = 0x5F3759DF - (i >> 1)
    yv = plsc.bitcast(i, jnp.float32)
    for _ in range(3):
        yv = yv * (1.5 - 0.5 * d2 * yv * yv)
    inv = yv
    ux = x * inv
    uy = y * inv
    uz = z * inv
    s = d2 * (1.0 / (R_CUT * R_CUT))
    c = jnp.float32(_COS_COEF[6])
    for k in (5, 4, 3, 2, 1, 0):
        c = c * s + jnp.float32(_COS_COEF[k])
    phi = jnp.where(s < 1.0, 0.5 * (c + 1.0), jnp.float32(0.0))
    return (
        _C0 * phi,
        (_C1 * phi) * ux, (_C1 * phi) * uy, (_C1 * phi) * uz,
        (_C2A * phi) * ux * uy, (_C2A * phi) * uy * uz,
        (_C2B * phi) * (3.0 * uz * uz - 1.0), (_C2A * phi) * ux * uz,
        (_C2C * phi) * (ux * ux - uy * uy),
    )


def _chi_sc_call(r2d, idx2d, n_nodes, n_edges):
    rows_per_sub = ((n_nodes + 16 * 8 - 1) // (16 * 8)) * 8
    n_pad = rows_per_sub * 16
    mesh = plsc.VectorSubcoreMesh(core_axis_name="c", subcore_axis_name="s")
    n_full = n_edges // 1024
    tail_edges = n_edges - n_full * 1024
    tail_inner = tail_edges // 128
    n_chunks = n_full + (1 if tail_edges else 0)
    iters = (n_chunks + 31) // 32

    @functools.partial(
        pl.kernel,
        mesh=mesh,
        out_type=jax.ShapeDtypeStruct((2, n_pad, 16), jnp.float32),
        scratch_types=[
            pltpu.VMEM((8, 128), jnp.int32),
            pltpu.VMEM((24, 128), jnp.float32),
            pltpu.VMEM((128, 16), jnp.float32),
            pltpu.VMEM((rows_per_sub, 16), jnp.float32),
            pltpu.VMEM_SHARED((n_pad, 16), jnp.float32),
        ],
        compiler_params=pltpu.CompilerParams(needs_layout_passes=False),
    )
    def chi_kernel(r_hbm, idx_hbm, out_hbm, idxv, rv, sv, tmp, acc):
        cid = lax.axis_index("c")
        sid = lax.axis_index("s")
        wid = sid * 2 + cid

        zero16 = jnp.zeros((16,), jnp.float32)

        def zrow(i, _):
            tmp[i, :] = zero16
            return 0

        lax.fori_loop(0, rows_per_sub, zrow, 0)

        def zrow2(i, _):
            sv[i, :] = zero16
            return 0

        lax.fori_loop(0, 128, zrow2, 0)

        r0 = sid * rows_per_sub
        pltpu.sync_copy(tmp, acc.at[pl.ds(r0, rows_per_sub)])
        plsc.subcore_barrier()

        lanes = lax.iota(jnp.int32, 16)

        def inner(k):
            for gidx in range(8):
                el = gidx * 16 + lanes
                p0 = (384 * k) + el * 3
                x = plsc.load_gather(rv, [p0 >> 7, p0 & 127])
                y = plsc.load_gather(rv, [(p0 + 1) >> 7, (p0 + 1) & 127])
                z = plsc.load_gather(rv, [(p0 + 2) >> 7, (p0 + 2) & 127])
                w = _edge_weights(x, y, z)
                for j in range(9):
                    plsc.store_scatter(
                        sv, [el, jnp.full((16,), j, jnp.int32)], w[j])
            pltpu.sync_copy(sv, acc.at[idxv.at[k]], add=True)

        def chunk_body(i, _):
            c = i * 32 + wid

            @pl.when(c < n_full)
            def _():
                pltpu.sync_copy(idx_hbm.at[pl.ds(8 * c, 8)], idxv)
                pltpu.sync_copy(r_hbm.at[pl.ds(24 * c, 24)], rv)
                for k in range(8):
                    inner(k)

            if tail_edges:
                @pl.when(c == n_full)
                def _():
                    pltpu.sync_copy(
                        idx_hbm.at[pl.ds(8 * n_full, tail_edges // 128)],
                        idxv.at[pl.ds(0, tail_edges // 128)])
                    pltpu.sync_copy(
                        r_hbm.at[pl.ds(24 * n_full, 3 * tail_edges // 128)],
                        rv.at[pl.ds(0, 3 * tail_edges // 128)])
                    for k in range(tail_inner):
                        inner(k)

            return 0

        lax.fori_loop(0, iters, chunk_body, 0)

        plsc.subcore_barrier()
        pltpu.sync_copy(acc.at[pl.ds(r0, rows_per_sub)], tmp)
        pltpu.sync_copy(tmp, out_hbm.at[cid, pl.ds(r0, rows_per_sub)])

    return chi_kernel(r2d, idx2d)


def _combine(partials, n_nodes):
    def body(p_ref, chi_ref):
        chi_ref[...] = (
            p_ref[0, :n_nodes, :9] + p_ref[1, :n_nodes, :9]
        ) * (1.0 / SPHC_NORM)

    return pl.pallas_call(
        body,
        out_shape=jax.ShapeDtypeStruct((n_nodes, 9), jnp.float32),
    )(partials)


def kernel(pairwise_distances, idx_i, z):
    n_edges = pairwise_distances.shape[0]
    n_nodes = z.shape[0]
    n_chunks = n_edges // 128

    rt = pairwise_distances.T
    sph_t, g_t = _tc_main(rt, n_edges)
    sph = sph_t.T
    g = jnp.transpose(g_t, (2, 0, 1))

    _BISECT_TC_ONLY = True
    if _BISECT_TC_ONLY:
        d_ij = jnp.linalg.norm(pairwise_distances, axis=-1)
        phi = jnp.where(d_ij < R_CUT,
                        0.5 * (jnp.cos(jnp.pi * d_ij / R_CUT) + 1.0), 0.0)
        chi = jax.ops.segment_sum(
            sph * phi[:, None], idx_i, num_segments=n_nodes) / SPHC_NORM
    else:
        r2d = pairwise_distances.reshape(n_edges * 3 // 128, 128)
        idx2d = idx_i.astype(jnp.int32).reshape(n_chunks, 128)
        partials = _chi_sc_call(r2d, idx2d, n_nodes, n_edges)
        chi = _combine(partials, n_nodes)
    return sph, chi, g
